# initial kernel scaffold (unmeasured)
import jax
import jax.numpy as jnp
from jax import lax
from jax.experimental import pallas as pl
from jax.experimental.pallas import tpu as pltpu

N_DEV = 4
SQ = 1024
SKV = 1024
HB = 8
DH = 128
DM = 1024
SCALE = 0.08838834764831843
NEG = -1e9


def kernel(x, Wq, K_ext, V_ext, Wo):
    x2 = x[0]
    Wq3 = Wq.reshape(DM, HB, DH).transpose(1, 0, 2)
    Wo3 = Wo.reshape(HB, DH, DM)
    K3 = K_ext[0].transpose(1, 0, 2)
    V3 = V_ext[0].transpose(1, 0, 2)

    def body(x_ref, wq_ref, k_ref, v_ref, wo_ref, out_ref,
             cwq_ref, cwo_ref, swq, rwq, swo, rwo):
        my_pos = lax.axis_index("i")
        left = lax.rem(my_pos + N_DEV - 1, N_DEV)
        right = lax.rem(my_pos + 1, N_DEV)

        barrier_sem = pltpu.get_barrier_semaphore()
        for nbr in (left, right):
            pl.semaphore_signal(barrier_sem, inc=1, device_id=(nbr,),
                                device_id_type=pl.DeviceIdType.MESH)
        pl.semaphore_wait(barrier_sem, 2)

        cwq_ref[0] = wq_ref[...]
        cwo_ref[0] = wo_ref[...]

        rows = lax.broadcasted_iota(jnp.int32, (SQ, SKV), 0) + my_pos * SQ
        cols = lax.broadcasted_iota(jnp.int32, (SQ, SKV), 1)
        qb = rows // 64
        kb = cols // 64
        mask = (qb == kb) | (kb == 0) | (lax.rem(qb + kb, 3) == 0)

        out_ref[0] = jnp.zeros((SQ, DM), jnp.float32)

        for h in range(N_DEV):
            if h < N_DEV - 1:
                rq = pltpu.make_async_remote_copy(
                    src_ref=cwq_ref.at[h], dst_ref=cwq_ref.at[h + 1],
                    send_sem=swq.at[h], recv_sem=rwq.at[h],
                    device_id=(right,), device_id_type=pl.DeviceIdType.MESH)
                ro = pltpu.make_async_remote_copy(
                    src_ref=cwo_ref.at[h], dst_ref=cwo_ref.at[h + 1],
                    send_sem=swo.at[h], recv_sem=rwo.at[h],
                    device_id=(right,), device_id_type=pl.DeviceIdType.MESH)
                rq.start()
                ro.start()

            j = lax.rem(my_pos - h + N_DEV, N_DEV)

            def head_step(hh, _, h=h, j=j):
                head = j * HB + hh
                q = jnp.dot(x_ref[...], cwq_ref[h, hh],
                            preferred_element_type=jnp.float32)
                s = lax.dot_general(q, k_ref[head],
                                    (((1,), (1,)), ((), ())),
                                    preferred_element_type=jnp.float32) * SCALE
                s = jnp.where(mask, s, NEG)
                m = jnp.max(s, axis=-1, keepdims=True)
                w = jnp.exp(s - m)
                w = w / jnp.sum(w, axis=-1, keepdims=True)
                ctx = jnp.dot(w, v_ref[head],
                              preferred_element_type=jnp.float32)
                out_ref[0] = out_ref[0] + jnp.dot(
                    ctx, cwo_ref[h, hh], preferred_element_type=jnp.float32)
                return 0

            lax.fori_loop(0, HB, head_step, 0)

            if h < N_DEV - 1:
                rq.wait()
                ro.wait()

    out = pl.pallas_call(
        body,
        out_shape=jax.ShapeDtypeStruct((1, SQ, DM), jnp.float32),
        in_specs=[pl.BlockSpec(memory_space=pltpu.VMEM)] * 5,
        out_specs=pl.BlockSpec(memory_space=pltpu.VMEM),
        scratch_shapes=[
            pltpu.VMEM((N_DEV, HB, DM, DH), jnp.float32),
            pltpu.VMEM((N_DEV, HB, DH, DM), jnp.float32),
            pltpu.SemaphoreType.DMA((N_DEV - 1,)),
            pltpu.SemaphoreType.DMA((N_DEV - 1,)),
            pltpu.SemaphoreType.DMA((N_DEV - 1,)),
            pltpu.SemaphoreType.DMA((N_DEV - 1,)),
        ],
        compiler_params=pltpu.CompilerParams(collective_id=0),
    )(x2, Wq3, K3, V3, Wo3)
    return out


# baseline (device time: 369758 ns/iter reference)
import jax
import jax.numpy as jnp
from jax import lax
from jax.experimental import pallas as pl
from jax.experimental.pallas import tpu as pltpu

N_DEV = 4
SQ = 1024
SKV = 1024
HB = 8
DH = 128
DM = 1024
SCALE = 0.08838834764831843
NEG = -1e9


def kernel(x, Wq, K_ext, V_ext, Wo):
    x2 = x[0]
    Wq3 = Wq.reshape(DM, HB, DH).transpose(1, 0, 2)
    Wo3 = Wo.reshape(HB, DH, DM)
    K3 = K_ext[0].transpose(1, 0, 2)
    V3 = V_ext[0].transpose(1, 0, 2)

    def body(x_ref, wq_ref, k_ref, v_ref, wo_ref, out_ref,
             cwq_ref, cwo_ref, kbuf_ref, vbuf_ref, bias_ref,
             swq, rwq, swo, rwo, ksem, vsem, wsem, credit_sem):
        my_pos = lax.axis_index("i")
        left = lax.rem(my_pos + N_DEV - 1, N_DEV)
        right = lax.rem(my_pos + 1, N_DEV)

        wq_cp = pltpu.make_async_copy(wq_ref, cwq_ref.at[0], wsem)
        wq_cp.start()
        wo_cp = pltpu.make_async_copy(wo_ref, cwo_ref.at[0], wsem)
        wo_cp.start()

        barrier_sem = pltpu.get_barrier_semaphore()
        for nbr in (left, right):
            pl.semaphore_signal(barrier_sem, inc=1, device_id=(nbr,),
                                device_id_type=pl.DeviceIdType.MESH)
        pl.semaphore_wait(barrier_sem, 2)
        wq_cp.wait()
        wo_cp.wait()

        rows = lax.broadcasted_iota(jnp.int32, (SQ, SKV), 0) + my_pos * SQ
        cols = lax.broadcasted_iota(jnp.int32, (SQ, SKV), 1)
        qb = rows // 64
        kb = cols // 64
        mask = (qb == kb) | (kb == 0) | (lax.rem(qb + kb, 3) == 0)
        bias_ref[...] = jnp.where(mask, 0.0, NEG)

        out_ref[0] = jnp.zeros((SQ, DM), jnp.float32)

        for h in range(N_DEV):
            if 1 <= h < N_DEV - 1:
                pl.semaphore_wait(credit_sem, 1)
            if h < N_DEV - 1:
                rq = pltpu.make_async_remote_copy(
                    src_ref=cwq_ref.at[h % 2], dst_ref=cwq_ref.at[(h + 1) % 2],
                    send_sem=swq.at[h], recv_sem=rwq.at[h],
                    device_id=(right,), device_id_type=pl.DeviceIdType.MESH)
                ro = pltpu.make_async_remote_copy(
                    src_ref=cwo_ref.at[h % 2], dst_ref=cwo_ref.at[(h + 1) % 2],
                    send_sem=swo.at[h], recv_sem=rwo.at[h],
                    device_id=(right,), device_id_type=pl.DeviceIdType.MESH)
                rq.start()
                ro.start()

            j = lax.rem(my_pos - h + N_DEV, N_DEV)

            def head_step(hh, _, h=h, j=j):
                head = j * HB + hh
                k_cp = pltpu.make_async_copy(k_ref.at[head], kbuf_ref, ksem)
                k_cp.start()
                v_cp = pltpu.make_async_copy(v_ref.at[head], vbuf_ref, vsem)
                v_cp.start()
                q = jnp.dot(x_ref[...], cwq_ref[h % 2, hh],
                            preferred_element_type=jnp.float32)
                k_cp.wait()
                s = lax.dot_general(q, kbuf_ref[...],
                                    (((1,), (1,)), ((), ())),
                                    preferred_element_type=jnp.float32)
                s = s * SCALE + bias_ref[...]
                m = jnp.max(s, axis=-1, keepdims=True)
                w = jnp.exp(s - m)
                w = w / jnp.sum(w, axis=-1, keepdims=True)
                v_cp.wait()
                ctx = jnp.dot(w, vbuf_ref[...],
                              preferred_element_type=jnp.float32)
                out_ref[0] = out_ref[0] + jnp.dot(
                    ctx, cwo_ref[h % 2, hh], preferred_element_type=jnp.float32)
                return 0

            lax.fori_loop(0, HB, head_step, 0)

            if h < N_DEV - 2:
                pl.semaphore_signal(credit_sem, inc=1, device_id=(left,),
                                    device_id_type=pl.DeviceIdType.MESH)
            if h < N_DEV - 1:
                rq.wait()
                ro.wait()

    out = pl.pallas_call(
        body,
        out_shape=jax.ShapeDtypeStruct((1, SQ, DM), jnp.float32),
        in_specs=[
            pl.BlockSpec(memory_space=pltpu.VMEM),
            pl.BlockSpec(memory_space=pl.ANY),
            pl.BlockSpec(memory_space=pl.ANY),
            pl.BlockSpec(memory_space=pl.ANY),
            pl.BlockSpec(memory_space=pl.ANY),
        ],
        out_specs=pl.BlockSpec(memory_space=pltpu.VMEM),
        scratch_shapes=[
            pltpu.VMEM((2, HB, DM, DH), jnp.float32),
            pltpu.VMEM((2, HB, DH, DM), jnp.float32),
            pltpu.VMEM((SKV, DH), jnp.float32),
            pltpu.VMEM((SKV, DH), jnp.float32),
            pltpu.VMEM((SQ, SKV), jnp.float32),
            pltpu.SemaphoreType.DMA((N_DEV - 1,)),
            pltpu.SemaphoreType.DMA((N_DEV - 1,)),
            pltpu.SemaphoreType.DMA((N_DEV - 1,)),
            pltpu.SemaphoreType.DMA((N_DEV - 1,)),
            pltpu.SemaphoreType.DMA,
            pltpu.SemaphoreType.DMA,
            pltpu.SemaphoreType.DMA,
            pltpu.SemaphoreType.REGULAR,
        ],
        compiler_params=pltpu.CompilerParams(collective_id=0),
    )(x2, Wq3, K3, V3, Wo3)
    return out


# device time: 255214 ns/iter; 1.4488x vs baseline; 1.4488x over previous
import jax
import jax.numpy as jnp
from jax import lax
from jax.experimental import pallas as pl
from jax.experimental.pallas import tpu as pltpu

N_DEV = 4
SQ = 1024
SKV = 1024
HB = 8
DH = 128
DM = 1024
SCALE = 0.08838834764831843
NEG = -1e9


def kernel(x, Wq, K_ext, V_ext, Wo):
    x2 = x[0]
    Wq3 = Wq.reshape(DM, HB, DH).transpose(1, 0, 2)
    Wo3 = Wo.reshape(HB, DH, DM)
    K3 = K_ext[0].transpose(1, 0, 2)
    V3 = V_ext[0].transpose(1, 0, 2)

    def body(x_ref, wq_ref, k_ref, v_ref, wo_ref, out_ref,
             cwq_ref, cwo_ref, kbuf_ref, vbuf_ref, bias_ref,
             swq, rwq, swo, rwo, ksem, vsem, wsem, credit_sem):
        my_pos = lax.axis_index("i")
        left = lax.rem(my_pos + N_DEV - 1, N_DEV)
        right = lax.rem(my_pos + 1, N_DEV)

        wq_cp = pltpu.make_async_copy(wq_ref, cwq_ref.at[0], wsem)
        wq_cp.start()
        wo_cp = pltpu.make_async_copy(wo_ref, cwo_ref.at[0], wsem)
        wo_cp.start()

        barrier_sem = pltpu.get_barrier_semaphore()
        for nbr in (left, right):
            pl.semaphore_signal(barrier_sem, inc=1, device_id=(nbr,),
                                device_id_type=pl.DeviceIdType.MESH)
        pl.semaphore_wait(barrier_sem, 2)
        wq_cp.wait()
        wo_cp.wait()

        rows = lax.broadcasted_iota(jnp.int32, (SQ, SKV), 0) + my_pos * SQ
        cols = lax.broadcasted_iota(jnp.int32, (SQ, SKV), 1)
        qb = rows // 64
        kb = cols // 64
        mask = (qb == kb) | (kb == 0) | (lax.rem(qb + kb, 3) == 0)
        bias_ref[...] = jnp.where(mask, 0.0, NEG)

        out_ref[0] = jnp.zeros((SQ, DM), jnp.float32)

        for h in range(N_DEV):
            if 1 <= h < N_DEV - 1:
                pass
            if h < N_DEV - 1:
                rq = pltpu.make_async_remote_copy(
                    src_ref=cwq_ref.at[h % 2], dst_ref=cwq_ref.at[(h + 1) % 2],
                    send_sem=swq.at[h], recv_sem=rwq.at[h],
                    device_id=(right,), device_id_type=pl.DeviceIdType.MESH)
                ro = pltpu.make_async_remote_copy(
                    src_ref=cwo_ref.at[h % 2], dst_ref=cwo_ref.at[(h + 1) % 2],
                    send_sem=swo.at[h], recv_sem=rwo.at[h],
                    device_id=(right,), device_id_type=pl.DeviceIdType.MESH)
                pass

            j = lax.rem(my_pos - h + N_DEV, N_DEV)

            def head_step(hh, _, h=h, j=j):
                head = j * HB + hh
                k_cp = pltpu.make_async_copy(k_ref.at[head], kbuf_ref, ksem)
                k_cp.start()
                v_cp = pltpu.make_async_copy(v_ref.at[head], vbuf_ref, vsem)
                v_cp.start()
                q = jnp.dot(x_ref[...], cwq_ref[h % 2, hh],
                            preferred_element_type=jnp.float32)
                k_cp.wait()
                s = lax.dot_general(q, kbuf_ref[...],
                                    (((1,), (1,)), ((), ())),
                                    preferred_element_type=jnp.float32)
                s = s * SCALE + bias_ref[...]
                m = jnp.max(s, axis=-1, keepdims=True)
                w = jnp.exp(s - m)
                w = w / jnp.sum(w, axis=-1, keepdims=True)
                v_cp.wait()
                ctx = jnp.dot(w, vbuf_ref[...],
                              preferred_element_type=jnp.float32)
                out_ref[0] = out_ref[0] + jnp.dot(
                    ctx, cwo_ref[h % 2, hh], preferred_element_type=jnp.float32)
                return 0

            lax.fori_loop(0, HB, head_step, 0)

            if h < N_DEV - 2:
                pass
            if h < N_DEV - 1:
                pass

    out = pl.pallas_call(
        body,
        out_shape=jax.ShapeDtypeStruct((1, SQ, DM), jnp.float32),
        in_specs=[
            pl.BlockSpec(memory_space=pltpu.VMEM),
            pl.BlockSpec(memory_space=pl.ANY),
            pl.BlockSpec(memory_space=pl.ANY),
            pl.BlockSpec(memory_space=pl.ANY),
            pl.BlockSpec(memory_space=pl.ANY),
        ],
        out_specs=pl.BlockSpec(memory_space=pltpu.VMEM),
        scratch_shapes=[
            pltpu.VMEM((2, HB, DM, DH), jnp.float32),
            pltpu.VMEM((2, HB, DH, DM), jnp.float32),
            pltpu.VMEM((SKV, DH), jnp.float32),
            pltpu.VMEM((SKV, DH), jnp.float32),
            pltpu.VMEM((SQ, SKV), jnp.float32),
            pltpu.SemaphoreType.DMA((N_DEV - 1,)),
            pltpu.SemaphoreType.DMA((N_DEV - 1,)),
            pltpu.SemaphoreType.DMA((N_DEV - 1,)),
            pltpu.SemaphoreType.DMA((N_DEV - 1,)),
            pltpu.SemaphoreType.DMA,
            pltpu.SemaphoreType.DMA,
            pltpu.SemaphoreType.DMA,
            pltpu.SemaphoreType.REGULAR,
        ],
        compiler_params=pltpu.CompilerParams(collective_id=0),
    )(x2, Wq3, K3, V3, Wo3)
    return out


# device time: 239255 ns/iter; 1.5455x vs baseline; 1.0667x over previous
import jax
import jax.numpy as jnp
from jax import lax
from jax.experimental import pallas as pl
from jax.experimental.pallas import tpu as pltpu

N_DEV = 4
SQ = 1024
SKV = 1024
HB = 8
HH = HB // 2
DH = 128
DM = 1024
SCALE = 0.08838834764831843
NEG = -1e9


def kernel(x, Wq, K_ext, V_ext, Wo):
    x2 = x[0]
    Wq3 = Wq.reshape(DM, HB, DH).transpose(1, 0, 2)
    Wo3 = Wo.reshape(HB, DH, DM)
    K3 = K_ext[0].transpose(1, 0, 2)
    V3 = V_ext[0].transpose(1, 0, 2)

    def body(x_ref, wq_ref, k_ref, v_ref, wo_ref, out_ref,
             cwq_ref, cwo_ref, kbuf_ref, vbuf_ref, bias_ref,
             sems, ksem, vsem, wsem, credit_cw, credit_ccw):
        my_pos = lax.axis_index("i")
        left = lax.rem(my_pos + N_DEV - 1, N_DEV)
        right = lax.rem(my_pos + 1, N_DEV)

        wq_cp = pltpu.make_async_copy(wq_ref, cwq_ref.at[0], wsem)
        wq_cp.start()
        wo_cp = pltpu.make_async_copy(wo_ref, cwo_ref.at[0], wsem)
        wo_cp.start()

        barrier_sem = pltpu.get_barrier_semaphore()
        for nbr in (left, right):
            pl.semaphore_signal(barrier_sem, inc=1, device_id=(nbr,),
                                device_id_type=pl.DeviceIdType.MESH)
        pl.semaphore_wait(barrier_sem, 2)
        wq_cp.wait()
        wo_cp.wait()

        rows = lax.broadcasted_iota(jnp.int32, (SQ, SKV), 0) + my_pos * SQ
        cols = lax.broadcasted_iota(jnp.int32, (SQ, SKV), 1)
        qb = rows // 64
        kb = cols // 64
        mask = (qb == kb) | (kb == 0) | (lax.rem(qb + kb, 3) == 0)
        bias_ref[...] = jnp.where(mask, 0.0, NEG)

        out_ref[0] = jnp.zeros((SQ, DM), jnp.float32)

        for h in range(N_DEV):
            slot, nxt = h % 2, (h + 1) % 2
            if 1 <= h < N_DEV - 1:
                pl.semaphore_wait(credit_cw, 1)
                pl.semaphore_wait(credit_ccw, 1)
            if h < N_DEV - 1:
                rdmas = [
                    pltpu.make_async_remote_copy(
                        src_ref=cwq_ref.at[slot, :HH], dst_ref=cwq_ref.at[nxt, :HH],
                        send_sem=sems.at[h, 0, 0], recv_sem=sems.at[h, 0, 1],
                        device_id=(right,), device_id_type=pl.DeviceIdType.MESH),
                    pltpu.make_async_remote_copy(
                        src_ref=cwo_ref.at[slot, :HH], dst_ref=cwo_ref.at[nxt, :HH],
                        send_sem=sems.at[h, 1, 0], recv_sem=sems.at[h, 1, 1],
                        device_id=(right,), device_id_type=pl.DeviceIdType.MESH),
                    pltpu.make_async_remote_copy(
                        src_ref=cwq_ref.at[slot, HH:], dst_ref=cwq_ref.at[nxt, HH:],
                        send_sem=sems.at[h, 2, 0], recv_sem=sems.at[h, 2, 1],
                        device_id=(left,), device_id_type=pl.DeviceIdType.MESH),
                    pltpu.make_async_remote_copy(
                        src_ref=cwo_ref.at[slot, HH:], dst_ref=cwo_ref.at[nxt, HH:],
                        send_sem=sems.at[h, 3, 0], recv_sem=sems.at[h, 3, 1],
                        device_id=(left,), device_id_type=pl.DeviceIdType.MESH),
                ]
                for r in rdmas:
                    r.start()

            jr = lax.rem(my_pos - h + N_DEV, N_DEV)
            jl = lax.rem(my_pos + h, N_DEV)

            def head_step(s, _, slot=slot, jr=jr, jl=jl):
                head = jnp.where(s < HH, jr, jl) * HB + s
                k_cp = pltpu.make_async_copy(k_ref.at[head], kbuf_ref, ksem)
                k_cp.start()
                v_cp = pltpu.make_async_copy(v_ref.at[head], vbuf_ref, vsem)
                v_cp.start()
                q = jnp.dot(x_ref[...], cwq_ref[slot, s],
                            preferred_element_type=jnp.float32)
                k_cp.wait()
                sc = lax.dot_general(q, kbuf_ref[...],
                                     (((1,), (1,)), ((), ())),
                                     preferred_element_type=jnp.float32)
                w = jnp.exp(sc * SCALE + bias_ref[...])
                w = w / jnp.sum(w, axis=-1, keepdims=True)
                v_cp.wait()
                ctx = jnp.dot(w, vbuf_ref[...],
                              preferred_element_type=jnp.float32)
                out_ref[0] = out_ref[0] + jnp.dot(
                    ctx, cwo_ref[slot, s], preferred_element_type=jnp.float32)
                return 0

            lax.fori_loop(0, HB, head_step, 0)

            if h < N_DEV - 2:
                pl.semaphore_signal(credit_cw, inc=1, device_id=(left,),
                                    device_id_type=pl.DeviceIdType.MESH)
                pl.semaphore_signal(credit_ccw, inc=1, device_id=(right,),
                                    device_id_type=pl.DeviceIdType.MESH)
            if h < N_DEV - 1:
                for r in rdmas:
                    r.wait()

    out = pl.pallas_call(
        body,
        out_shape=jax.ShapeDtypeStruct((1, SQ, DM), jnp.float32),
        in_specs=[
            pl.BlockSpec(memory_space=pltpu.VMEM),
            pl.BlockSpec(memory_space=pl.ANY),
            pl.BlockSpec(memory_space=pl.ANY),
            pl.BlockSpec(memory_space=pl.ANY),
            pl.BlockSpec(memory_space=pl.ANY),
        ],
        out_specs=pl.BlockSpec(memory_space=pltpu.VMEM),
        scratch_shapes=[
            pltpu.VMEM((2, HB, DM, DH), jnp.float32),
            pltpu.VMEM((2, HB, DH, DM), jnp.float32),
            pltpu.VMEM((SKV, DH), jnp.float32),
            pltpu.VMEM((SKV, DH), jnp.float32),
            pltpu.VMEM((SQ, SKV), jnp.float32),
            pltpu.SemaphoreType.DMA((N_DEV - 1, 4, 2)),
            pltpu.SemaphoreType.DMA,
            pltpu.SemaphoreType.DMA,
            pltpu.SemaphoreType.DMA,
            pltpu.SemaphoreType.REGULAR,
            pltpu.SemaphoreType.REGULAR,
        ],
        compiler_params=pltpu.CompilerParams(collective_id=0),
    )(x2, Wq3, K3, V3, Wo3)
    return out


# device time: 153568 ns/iter; 2.4078x vs baseline; 1.5580x over previous
import jax
import jax.numpy as jnp
from jax import lax
from jax.experimental import pallas as pl
from jax.experimental.pallas import tpu as pltpu

N_DEV = 4
SQ = 1024
SKV = 1024
HB = 8
HH = HB // 2
DH = 128
DM = 1024
SCALE = 0.08838834764831843
NEG = -1e9


def kernel(x, Wq, K_ext, V_ext, Wo):
    x2 = x[0].astype(jnp.bfloat16)
    Wq3 = (Wq * SCALE).reshape(DM, HB, DH).transpose(1, 0, 2).astype(jnp.bfloat16)
    Wo3 = Wo.reshape(HB, DH, DM).astype(jnp.bfloat16)
    K3 = K_ext[0].transpose(1, 0, 2).astype(jnp.bfloat16)
    V3 = V_ext[0].transpose(1, 0, 2).astype(jnp.bfloat16)

    def body(x_ref, wq_ref, k_ref, v_ref, wo_ref, out_ref,
             cwq_ref, cwo_ref, bias_ref,
             sems, wsem, credit_cw, credit_ccw):
        my_pos = lax.axis_index("i")
        left = lax.rem(my_pos + N_DEV - 1, N_DEV)
        right = lax.rem(my_pos + 1, N_DEV)

        wq_cp = pltpu.make_async_copy(wq_ref, cwq_ref.at[0], wsem)
        wq_cp.start()
        wo_cp = pltpu.make_async_copy(wo_ref, cwo_ref.at[0], wsem)
        wo_cp.start()

        barrier_sem = pltpu.get_barrier_semaphore()
        for nbr in (left, right):
            pl.semaphore_signal(barrier_sem, inc=1, device_id=(nbr,),
                                device_id_type=pl.DeviceIdType.MESH)
        pl.semaphore_wait(barrier_sem, 2)
        wq_cp.wait()
        wo_cp.wait()

        rows = lax.broadcasted_iota(jnp.int32, (SQ, SKV), 0) + my_pos * SQ
        cols = lax.broadcasted_iota(jnp.int32, (SQ, SKV), 1)
        qb = rows // 64
        kb = cols // 64
        mask = (qb == kb) | (kb == 0) | (lax.rem(qb + kb, 3) == 0)
        bias_ref[...] = jnp.where(mask, 0.0, NEG)

        out_ref[0] = jnp.zeros((SQ, DM), jnp.float32)

        for h in range(N_DEV):
            slot, nxt = h % 2, (h + 1) % 2
            if 1 <= h < N_DEV - 1:
                pl.semaphore_wait(credit_cw, 1)
                pl.semaphore_wait(credit_ccw, 1)
            if h < N_DEV - 1:
                rdmas = [
                    pltpu.make_async_remote_copy(
                        src_ref=cwq_ref.at[slot, :HH], dst_ref=cwq_ref.at[nxt, :HH],
                        send_sem=sems.at[h, 0, 0], recv_sem=sems.at[h, 0, 1],
                        device_id=(right,), device_id_type=pl.DeviceIdType.MESH),
                    pltpu.make_async_remote_copy(
                        src_ref=cwo_ref.at[slot, :HH], dst_ref=cwo_ref.at[nxt, :HH],
                        send_sem=sems.at[h, 1, 0], recv_sem=sems.at[h, 1, 1],
                        device_id=(right,), device_id_type=pl.DeviceIdType.MESH),
                    pltpu.make_async_remote_copy(
                        src_ref=cwq_ref.at[slot, HH:], dst_ref=cwq_ref.at[nxt, HH:],
                        send_sem=sems.at[h, 2, 0], recv_sem=sems.at[h, 2, 1],
                        device_id=(left,), device_id_type=pl.DeviceIdType.MESH),
                    pltpu.make_async_remote_copy(
                        src_ref=cwo_ref.at[slot, HH:], dst_ref=cwo_ref.at[nxt, HH:],
                        send_sem=sems.at[h, 3, 0], recv_sem=sems.at[h, 3, 1],
                        device_id=(left,), device_id_type=pl.DeviceIdType.MESH),
                ]
                for r in rdmas:
                    r.start()

            jr = lax.rem(my_pos - h + N_DEV, N_DEV)
            jl = lax.rem(my_pos + h, N_DEV)

            def head_step(s, _, slot=slot, jr=jr, jl=jl):
                head = jnp.where(s < HH, jr, jl) * HB + s
                q = jnp.dot(x_ref[...], cwq_ref[slot, s],
                            preferred_element_type=jnp.float32)
                sc = lax.dot_general(q.astype(jnp.bfloat16), k_ref[head],
                                     (((1,), (1,)), ((), ())),
                                     preferred_element_type=jnp.float32)
                w = jnp.exp(sc + bias_ref[...])
                denom = jnp.sum(w, axis=-1, keepdims=True)
                ctx = jnp.dot(w.astype(jnp.bfloat16), v_ref[head],
                              preferred_element_type=jnp.float32) / denom
                out_ref[0] = out_ref[0] + jnp.dot(
                    ctx.astype(jnp.bfloat16), cwo_ref[slot, s],
                    preferred_element_type=jnp.float32)
                return 0

            lax.fori_loop(0, HB, head_step, 0)

            if h < N_DEV - 2:
                pl.semaphore_signal(credit_cw, inc=1, device_id=(left,),
                                    device_id_type=pl.DeviceIdType.MESH)
                pl.semaphore_signal(credit_ccw, inc=1, device_id=(right,),
                                    device_id_type=pl.DeviceIdType.MESH)
            if h < N_DEV - 1:
                for r in rdmas:
                    r.wait()

    out = pl.pallas_call(
        body,
        out_shape=jax.ShapeDtypeStruct((1, SQ, DM), jnp.float32),
        in_specs=[pl.BlockSpec(memory_space=pltpu.VMEM)] * 5,
        out_specs=pl.BlockSpec(memory_space=pltpu.VMEM),
        scratch_shapes=[
            pltpu.VMEM((2, HB, DM, DH), jnp.bfloat16),
            pltpu.VMEM((2, HB, DH, DM), jnp.bfloat16),
            pltpu.VMEM((SQ, SKV), jnp.float32),
            pltpu.SemaphoreType.DMA((N_DEV - 1, 4, 2)),
            pltpu.SemaphoreType.DMA,
            pltpu.SemaphoreType.REGULAR,
            pltpu.SemaphoreType.REGULAR,
        ],
        compiler_params=pltpu.CompilerParams(collective_id=0),
    )(x2, Wq3, K3, V3, Wo3)
    return out


# device time: 94064 ns/iter; 3.9309x vs baseline; 1.6326x over previous
import jax
import jax.numpy as jnp
from jax import lax
from jax.experimental import pallas as pl
from jax.experimental.pallas import tpu as pltpu

N_DEV = 4
SQ = 1024
SKV = 1024
HB = 8
HH = HB // 2
DH = 128
DM = 1024
HW = HH * DH
SCALE = 0.08838834764831843
NEG = -1e9


def kernel(x, Wq, K_ext, V_ext, Wo):
    x2 = x[0].astype(jnp.bfloat16)
    Wqs = (Wq * SCALE).astype(jnp.bfloat16)
    Wos = Wo.astype(jnp.bfloat16)
    Kb = K_ext[0]
    Vb = V_ext[0]

    def body(xb_ref, wqb_ref, k_ref, v_ref, wob_ref, out_ref,
             qcw_ref, qccw_ref, ocw_ref, occw_ref,
             kcw_ref, kccw_ref, vcw_ref, vccw_ref,
             qscw_ref, qsccw_ref, bias_ref, wbuf_ref,
             ctxscw_ref, ctxsccw_ref, ctxfcw_ref, ctxfccw_ref,
             sems, ksems, vsems, credit_cw, credit_ccw):
        my_pos = lax.axis_index("i")
        left = lax.rem(my_pos + N_DEV - 1, N_DEV)
        right = lax.rem(my_pos + 1, N_DEV)

        def k_fetch(slot, jr, jl):
            cps = []
            for s in range(HH):
                cps += [
                    pltpu.make_async_copy(
                        k_ref.at[:, jr * HB + s, :],
                        kcw_ref.at[slot, s], ksems.at[slot, 0, s]),
                    pltpu.make_async_copy(
                        k_ref.at[:, jl * HB + HH + s, :],
                        kccw_ref.at[slot, s], ksems.at[slot, 1, s]),
                ]
            for c in cps:
                c.start()
            return cps

        def v_fetch(jr, jl):
            cps = []
            for s in range(HH):
                cps += [
                    pltpu.make_async_copy(
                        v_ref.at[:, jr * HB + s, :],
                        vcw_ref.at[s], vsems.at[0, s]),
                    pltpu.make_async_copy(
                        v_ref.at[:, jl * HB + HH + s, :],
                        vccw_ref.at[s], vsems.at[1, s]),
                ]
            for c in cps:
                c.start()
            return cps

        kv_pending = {0: k_fetch(0, my_pos, my_pos) + v_fetch(my_pos, my_pos)}

        barrier_sem = pltpu.get_barrier_semaphore()
        for nbr in (left, right):
            pl.semaphore_signal(barrier_sem, inc=1, device_id=(nbr,),
                                device_id_type=pl.DeviceIdType.MESH)

        qcw_ref[0] = wqb_ref[:, :HW]
        qccw_ref[0] = wqb_ref[:, HW:]
        ocw_ref[0] = wob_ref[:HW, :]
        occw_ref[0] = wob_ref[HW:, :]

        rows = lax.broadcasted_iota(jnp.int32, (SQ, SKV), 0) + my_pos * SQ
        cols = lax.broadcasted_iota(jnp.int32, (SQ, SKV), 1)
        qb = rows // 64
        kb = cols // 64
        mask = (qb == kb) | (kb == 0) | (lax.rem(qb + kb, 3) == 0)
        bias_ref[...] = jnp.where(mask, 0.0, NEG).astype(jnp.bfloat16)

        pl.semaphore_wait(barrier_sem, 2)

        for h in range(N_DEV):
            slot, nxt = h % 2, (h + 1) % 2
            if 1 <= h < N_DEV - 1:
                pl.semaphore_wait(credit_cw, 1)
                pl.semaphore_wait(credit_ccw, 1)
            if h >= 1:
                for r in pending_wq:
                    r.wait()
            if h < N_DEV - 1:
                pending_wq = [
                    pltpu.make_async_remote_copy(
                        src_ref=qcw_ref.at[slot], dst_ref=qcw_ref.at[nxt],
                        send_sem=sems.at[h, 0, 0], recv_sem=sems.at[h, 0, 1],
                        device_id=(right,), device_id_type=pl.DeviceIdType.MESH),
                    pltpu.make_async_remote_copy(
                        src_ref=qccw_ref.at[slot], dst_ref=qccw_ref.at[nxt],
                        send_sem=sems.at[h, 2, 0], recv_sem=sems.at[h, 2, 1],
                        device_id=(left,), device_id_type=pl.DeviceIdType.MESH),
                ]
                for r in pending_wq:
                    r.start()

            for c in kv_pending.pop(h):
                c.wait()
            if h < N_DEV - 1:
                next_k = k_fetch(nxt,
                                 lax.rem(my_pos - h - 1 + N_DEV, N_DEV),
                                 lax.rem(my_pos + h + 1, N_DEV))

            q_cw = jnp.dot(xb_ref[...], qcw_ref[slot],
                           preferred_element_type=jnp.float32)
            q_ccw = jnp.dot(xb_ref[...], qccw_ref[slot],
                            preferred_element_type=jnp.float32)
            for s in range(HH):
                c0, c1 = s * DH, (s + 1) * DH
                qscw_ref[s] = q_cw[:, c0:c1].astype(jnp.bfloat16)
                qsccw_ref[s] = q_ccw[:, c0:c1].astype(jnp.bfloat16)

            for qs_ref, kh_ref, vh_ref, ctxs_ref in (
                    (qscw_ref, kcw_ref, vcw_ref, ctxscw_ref),
                    (qsccw_ref, kccw_ref, vccw_ref, ctxsccw_ref)):

                def head_step(s, _, qs_ref=qs_ref, kh_ref=kh_ref,
                              vh_ref=vh_ref, ctxs_ref=ctxs_ref, slot=slot):
                    sc = lax.dot_general(qs_ref[s],
                                         kh_ref[slot, s].astype(jnp.bfloat16),
                                         (((1,), (1,)), ((), ())),
                                         preferred_element_type=jnp.float32)
                    wf = jnp.exp(sc + bias_ref[...].astype(jnp.float32))
                    denom = jnp.sum(wf, axis=-1, keepdims=True)
                    wbuf_ref[...] = wf.astype(jnp.bfloat16)
                    ctx = jnp.dot(wbuf_ref[...],
                                  vh_ref[s].astype(jnp.bfloat16),
                                  preferred_element_type=jnp.float32) / denom
                    ctxs_ref[s] = ctx.astype(jnp.bfloat16)
                    return 0

                lax.fori_loop(0, HH, head_step, 0)

            if h >= 1:
                for r in pending_wo:
                    r.wait()
            for s in range(HH):
                c0, c1 = s * DH, (s + 1) * DH
                ctxfcw_ref[:, c0:c1] = ctxscw_ref[s]
                ctxfccw_ref[:, c0:c1] = ctxsccw_ref[s]
            part_cw = jnp.dot(ctxfcw_ref[...], ocw_ref[slot],
                              preferred_element_type=jnp.float32)
            if h == 0:
                out_ref[0] = part_cw
            else:
                out_ref[0] = out_ref[0] + part_cw
            out_ref[0] = out_ref[0] + jnp.dot(
                ctxfccw_ref[...], occw_ref[slot],
                preferred_element_type=jnp.float32)

            if h < N_DEV - 1:
                pending_wo = [
                    pltpu.make_async_remote_copy(
                        src_ref=ocw_ref.at[slot], dst_ref=ocw_ref.at[nxt],
                        send_sem=sems.at[h, 1, 0], recv_sem=sems.at[h, 1, 1],
                        device_id=(right,), device_id_type=pl.DeviceIdType.MESH),
                    pltpu.make_async_remote_copy(
                        src_ref=occw_ref.at[slot], dst_ref=occw_ref.at[nxt],
                        send_sem=sems.at[h, 3, 0], recv_sem=sems.at[h, 3, 1],
                        device_id=(left,), device_id_type=pl.DeviceIdType.MESH),
                ]
                for r in pending_wo:
                    r.start()
                kv_pending[h + 1] = next_k + v_fetch(
                    lax.rem(my_pos - h - 1 + N_DEV, N_DEV),
                    lax.rem(my_pos + h + 1, N_DEV))

            if h < N_DEV - 2:
                pl.semaphore_signal(credit_cw, inc=1, device_id=(left,),
                                    device_id_type=pl.DeviceIdType.MESH)
                pl.semaphore_signal(credit_ccw, inc=1, device_id=(right,),
                                    device_id_type=pl.DeviceIdType.MESH)

    out = pl.pallas_call(
        body,
        out_shape=jax.ShapeDtypeStruct((1, SQ, DM), jnp.float32),
        in_specs=[
            pl.BlockSpec(memory_space=pltpu.VMEM),
            pl.BlockSpec(memory_space=pltpu.VMEM),
            pl.BlockSpec(memory_space=pl.ANY),
            pl.BlockSpec(memory_space=pl.ANY),
            pl.BlockSpec(memory_space=pltpu.VMEM),
        ],
        out_specs=pl.BlockSpec(memory_space=pltpu.VMEM),
        scratch_shapes=[
            pltpu.VMEM((2, DM, HW), jnp.bfloat16),
            pltpu.VMEM((2, DM, HW), jnp.bfloat16),
            pltpu.VMEM((2, HW, DM), jnp.bfloat16),
            pltpu.VMEM((2, HW, DM), jnp.bfloat16),
            pltpu.VMEM((2, HH, SKV, DH), jnp.float32),
            pltpu.VMEM((2, HH, SKV, DH), jnp.float32),
            pltpu.VMEM((HH, SKV, DH), jnp.float32),
            pltpu.VMEM((HH, SKV, DH), jnp.float32),
            pltpu.VMEM((HH, SQ, DH), jnp.bfloat16),
            pltpu.VMEM((HH, SQ, DH), jnp.bfloat16),
            pltpu.VMEM((SQ, SKV), jnp.bfloat16),
            pltpu.VMEM((SQ, SKV), jnp.bfloat16),
            pltpu.VMEM((HH, SQ, DH), jnp.bfloat16),
            pltpu.VMEM((HH, SQ, DH), jnp.bfloat16),
            pltpu.VMEM((SQ, HW), jnp.bfloat16),
            pltpu.VMEM((SQ, HW), jnp.bfloat16),
            pltpu.SemaphoreType.DMA((N_DEV - 1, 4, 2)),
            pltpu.SemaphoreType.DMA((2, 2, HH)),
            pltpu.SemaphoreType.DMA((2, HH)),
            pltpu.SemaphoreType.REGULAR,
            pltpu.SemaphoreType.REGULAR,
        ],
        compiler_params=pltpu.CompilerParams(collective_id=0),
    )(x2, Wqs, Kb, Vb, Wos)
    return out
